# baseline (speedup 1.0000x reference)
import functools

import jax
import jax.numpy as jnp
from jax import lax
from jax.experimental import pallas as pl
from jax.experimental.pallas import tpu as pltpu
from jax.experimental.pallas import tpu_sc as plsc

_B = 16384
_D = 64
_NW = 32
_BPW = _B // _NW
_CHUNK = 128
_NCH = _BPW // _CHUNK


def _sc_gather_body(uidx_hbm, iidx_hbm, uemb_hbm, iemb_hbm, u_out, v_out,
                    uidx_v, iidx_v, urows_v, irows_v, sem):
    wid = lax.axis_index("s") * 2 + lax.axis_index("c")
    base = wid * _BPW
    pltpu.sync_copy(uidx_hbm.at[wid], uidx_v)
    pltpu.sync_copy(iidx_hbm.at[wid], iidx_v)
    for j in range(_NCH):
        pltpu.async_copy(uemb_hbm.at[uidx_v.at[j]],
                         urows_v.at[pl.ds(j * _CHUNK, _CHUNK)], sem)
        pltpu.async_copy(iemb_hbm.at[iidx_v.at[j]],
                         irows_v.at[pl.ds(j * _CHUNK, _CHUNK)], sem)
    for j in range(_NCH):
        pltpu.make_async_copy(uemb_hbm.at[uidx_v.at[j]],
                              urows_v.at[pl.ds(j * _CHUNK, _CHUNK)], sem).wait()
        pltpu.make_async_copy(iemb_hbm.at[iidx_v.at[j]],
                              irows_v.at[pl.ds(j * _CHUNK, _CHUNK)], sem).wait()
    pltpu.sync_copy(urows_v, u_out.at[pl.ds(base, _BPW)])
    pltpu.sync_copy(irows_v, v_out.at[pl.ds(base, _BPW)])


@jax.jit
def _sc_gather(user_idx, item_idx, user_emb, item_emb):
    mesh = plsc.VectorSubcoreMesh(core_axis_name="c", subcore_axis_name="s")
    f = pl.kernel(
        _sc_gather_body,
        out_type=(
            jax.ShapeDtypeStruct((_B, _D), jnp.float32),
            jax.ShapeDtypeStruct((_B, _D), jnp.float32),
        ),
        mesh=mesh,
        scratch_types=[
            pltpu.VMEM((_NCH, _CHUNK), jnp.int32),
            pltpu.VMEM((_NCH, _CHUNK), jnp.int32),
            pltpu.VMEM((_BPW, _D), jnp.float32),
            pltpu.VMEM((_BPW, _D), jnp.float32),
            pltpu.SemaphoreType.DMA,
        ],
        compiler_params=pltpu.CompilerParams(use_tc_tiling_on_sc=False),
    )
    uidx3 = user_idx.reshape(_NW, _NCH, _CHUNK)
    iidx3 = item_idx.reshape(_NW, _NCH, _CHUNK)
    return f(uidx3, iidx3, user_emb, item_emb)


_BLK = 1024


def _mlp_body(u_ref, v_ref, w1u_ref, w1v_ref, b1_ref, w2_ref, b2_ref,
              w3_ref, b3_ref, wo_ref, bo_ref, out_ref):
    h = u_ref[...] @ w1u_ref[...] + v_ref[...] @ w1v_ref[...] + b1_ref[...]
    h = jnp.maximum(h, 0.0)
    h = jnp.maximum(h @ w2_ref[...] + b2_ref[...], 0.0)
    h = jnp.maximum(h @ w3_ref[...] + b3_ref[...], 0.0)
    o = jnp.sum(h * wo_ref[...], axis=1, keepdims=True) + bo_ref[0, 0]
    out_ref[...] = jnp.maximum(o, 0.0)


@jax.jit
def _tc_mlp(u, v, W1, b1, W2, b2, W3, b3, Wo, bo):
    rep = lambda s: pl.BlockSpec(s, lambda i: (0,) * len(s))
    f = pl.pallas_call(
        _mlp_body,
        grid=(_B // _BLK,),
        in_specs=[
            pl.BlockSpec((_BLK, _D), lambda i: (i, 0)),
            pl.BlockSpec((_BLK, _D), lambda i: (i, 0)),
            rep((_D, 256)), rep((_D, 256)), rep((1, 256)),
            rep((256, 128)), rep((1, 128)),
            rep((128, 64)), rep((1, 64)),
            rep((1, 64)), rep((1, 1)),
        ],
        out_specs=pl.BlockSpec((_BLK, 1), lambda i: (i, 0)),
        out_shape=jax.ShapeDtypeStruct((_B, 1), jnp.float32),
    )
    return f(u, v, W1[:_D], W1[_D:], b1.reshape(1, -1), W2, b2.reshape(1, -1),
             W3, b3.reshape(1, -1), Wo.reshape(1, -1), bo.reshape(1, 1))


def kernel(user_idx, item_idx, user_emb, item_emb,
           W1, b1, W2, b2, W3, b3, Wo, bo):
    u, v = _sc_gather(user_idx, item_idx, user_emb, item_emb)
    return _tc_mlp(u, v, W1, b1, W2, b2, W3, b3, Wo, bo)
